# Initial kernel scaffold; baseline (speedup 1.0000x reference)
import jax
import jax.numpy as jnp
from jax.experimental import pallas as pl


def kernel(LUT, x):
    raise NotImplementedError("write your pallas kernel here")


# speedup vs baseline: 131.0707x; 131.0707x over previous
import functools

import jax
import jax.numpy as jnp
import numpy as np
from jax import lax
from jax.experimental import pallas as pl
from jax.experimental.pallas import tpu as pltpu
from jax.experimental.pallas import tpu_sc as plsc

_DIM = 33
_N3 = _DIM * _DIM * _DIM
_N3P = 35944
_BINSIZE = np.float32(1.0001 / (_DIM - 1))
_INV_BIN = np.float32((_DIM - 1) / 1.0001)
_LANES = 16
_CHUNK = 1024
_NW = 32


def _build_sc_call(npix, plane):
    nbatch = npix // plane
    wper = _NW // nbatch
    q = plane // wper
    nchunks = q // _CHUNK
    ngroups = _CHUNK // _LANES

    mesh = plsc.VectorSubcoreMesh(core_axis_name="c", subcore_axis_name="s")

    @functools.partial(
        pl.kernel,
        out_type=jax.ShapeDtypeStruct((npix * 3,), jnp.float32),
        mesh=mesh,
        compiler_params=pltpu.CompilerParams(needs_layout_passes=False),
        scratch_types=[
            pltpu.VMEM((_N3P,), jnp.float32),
            pltpu.VMEM((_N3P,), jnp.float32),
            pltpu.VMEM((_N3P,), jnp.float32),
            pltpu.VMEM((_CHUNK,), jnp.float32),
            pltpu.VMEM((_CHUNK,), jnp.float32),
            pltpu.VMEM((_CHUNK,), jnp.float32),
            pltpu.VMEM((_CHUNK,), jnp.float32),
            pltpu.VMEM((_CHUNK,), jnp.float32),
            pltpu.VMEM((_CHUNK,), jnp.float32),
        ],
    )
    def sc_fn(lut_hbm, x_hbm, out_hbm, l0, l1, l2, rb, gb, bb, o0, o1, o2):
        cid = lax.axis_index("c")
        sid = lax.axis_index("s")
        wid = sid * 2 + cid
        batch = wid // wper
        quarter = wid % wper
        r_base = (batch * 3 + 0) * plane + quarter * q
        g_base = r_base + plane
        b_base = r_base + 2 * plane

        pltpu.sync_copy(lut_hbm.at[pl.ds(0, _N3P)], l0)
        pltpu.sync_copy(lut_hbm.at[pl.ds(_N3P, _N3P)], l1)
        pltpu.sync_copy(lut_hbm.at[pl.ds(2 * _N3P, _N3P)], l2)

        @pl.loop(0, nchunks)
        def _chunk(ci):
            off = ci * _CHUNK
            pltpu.sync_copy(x_hbm.at[pl.ds(r_base + off, _CHUNK)], rb)
            pltpu.sync_copy(x_hbm.at[pl.ds(g_base + off, _CHUNK)], gb)
            pltpu.sync_copy(x_hbm.at[pl.ds(b_base + off, _CHUNK)], bb)

            @pl.loop(0, ngroups)
            def _group(gi):
                s = pl.multiple_of(gi * _LANES, _LANES)
                r = rb[pl.ds(s, _LANES)]
                g = gb[pl.ds(s, _LANES)]
                b = bb[pl.ds(s, _LANES)]
                rid = (r * _INV_BIN).astype(jnp.int32)
                gid = (g * _INV_BIN).astype(jnp.int32)
                bid = (b * _INV_BIN).astype(jnp.int32)
                rd = (r - rid.astype(jnp.float32) * _BINSIZE) * _INV_BIN
                gd = (g - gid.astype(jnp.float32) * _BINSIZE) * _INV_BIN
                bd = (b - bid.astype(jnp.float32) * _BINSIZE) * _INV_BIN
                i000 = rid + gid * _DIM + bid * (_DIM * _DIM)
                i100 = i000 + 1
                i010 = i000 + _DIM
                i110 = i000 + _DIM + 1
                i001 = i000 + _DIM * _DIM
                i101 = i001 + 1
                i011 = i001 + _DIM
                i111 = i001 + _DIM + 1

                def interp(lref):
                    v000 = plsc.load_gather(lref, [i000])
                    v100 = plsc.load_gather(lref, [i100])
                    v010 = plsc.load_gather(lref, [i010])
                    v110 = plsc.load_gather(lref, [i110])
                    v001 = plsc.load_gather(lref, [i001])
                    v101 = plsc.load_gather(lref, [i101])
                    v011 = plsc.load_gather(lref, [i011])
                    v111 = plsc.load_gather(lref, [i111])
                    a = v000 + rd * (v100 - v000)
                    b_ = v010 + rd * (v110 - v010)
                    c_ = v001 + rd * (v101 - v001)
                    d_ = v011 + rd * (v111 - v011)
                    e = a + gd * (b_ - a)
                    f = c_ + gd * (d_ - c_)
                    return e + bd * (f - e)

                o0[pl.ds(s, _LANES)] = interp(l0)
                o1[pl.ds(s, _LANES)] = interp(l1)
                o2[pl.ds(s, _LANES)] = interp(l2)

            pltpu.sync_copy(o0, out_hbm.at[pl.ds(r_base + off, _CHUNK)])
            pltpu.sync_copy(o1, out_hbm.at[pl.ds(g_base + off, _CHUNK)])
            pltpu.sync_copy(o2, out_hbm.at[pl.ds(b_base + off, _CHUNK)])

    return sc_fn


def kernel(LUT, x):
    B, C, H, W = x.shape
    plane = H * W
    npix = B * plane
    lut_flat = jnp.pad(LUT.reshape(3, _N3), ((0, 0), (0, _N3P - _N3))).reshape(-1)
    x_flat = x.reshape(npix * 3)
    out_flat = _build_sc_call(npix, plane)(lut_flat, x_flat)
    return out_flat.reshape(B, C, H, W)


# speedup vs baseline: 300.4603x; 2.2924x over previous
import functools

import jax
import jax.numpy as jnp
import numpy as np
from jax import lax
from jax.experimental import pallas as pl
from jax.experimental.pallas import tpu as pltpu
from jax.experimental.pallas import tpu_sc as plsc

_DIM = 33
_N3 = _DIM * _DIM * _DIM
_N3P = 35944
_BINSIZE = np.float32(1.0001 / (_DIM - 1))
_INV_BIN = np.float32((_DIM - 1) / 1.0001)
_LANES = 16
_CHUNK = 1024
_NW = 32
_UNROLL = 4


def _build_sc_call(npix, plane):
    nbatch = npix // plane
    wper = _NW // nbatch
    q = plane // wper
    nchunks = q // _CHUNK

    mesh = plsc.VectorSubcoreMesh(core_axis_name="c", subcore_axis_name="s")

    @functools.partial(
        pl.kernel,
        out_type=jax.ShapeDtypeStruct((npix * 3,), jnp.float32),
        mesh=mesh,
        compiler_params=pltpu.CompilerParams(needs_layout_passes=False),
        scratch_types=(
            [pltpu.VMEM((3 * _N3,), jnp.float32)]
            + [pltpu.VMEM((_CHUNK,), jnp.float32)] * 12
            + [pltpu.SemaphoreType.DMA] * 4
        ),
    )
    def sc_fn(lut_hbm, x_hbm, out_hbm, lut, *rest):
        bufs = rest[:12]
        isems = rest[12:14]
        osems = rest[14:16]
        rbufs, gbufs, bbufs = bufs[0:2], bufs[2:4], bufs[4:6]
        obufs = [bufs[6:9], bufs[9:12]]

        cid = lax.axis_index("c")
        sid = lax.axis_index("s")
        wid = sid * 2 + cid
        batch = wid // wper
        quarter = wid % wper
        base0 = (batch * 3) * plane + quarter * q
        bases = [base0, base0 + plane, base0 + 2 * plane]

        pltpu.sync_copy(lut_hbm, lut)

        def start_in(chunk, slot):
            off = chunk * _CHUNK
            for c, buf in enumerate((rbufs[slot], gbufs[slot], bbufs[slot])):
                pltpu.async_copy(
                    x_hbm.at[pl.ds(bases[c] + off, _CHUNK)], buf, isems[slot]
                )

        def wait_in(slot):
            for buf in (rbufs[slot], gbufs[slot], bbufs[slot]):
                pltpu.make_async_copy(
                    x_hbm.at[pl.ds(0, _CHUNK)], buf, isems[slot]
                ).wait()

        def start_out(chunk, slot):
            off = chunk * _CHUNK
            for c in range(3):
                pltpu.async_copy(
                    obufs[slot][c],
                    out_hbm.at[pl.ds(bases[c] + off, _CHUNK)],
                    osems[slot],
                )

        def wait_out(slot):
            for c in range(3):
                pltpu.make_async_copy(
                    obufs[slot][c],
                    out_hbm.at[pl.ds(0, _CHUNK)],
                    osems[slot],
                ).wait()

        def compute(slot):
            rb, gb, bb = rbufs[slot], gbufs[slot], bbufs[slot]
            o0, o1, o2 = obufs[slot]

            @plsc.parallel_loop(0, _CHUNK, step=_LANES, unroll=_UNROLL)
            def _group(s):
                s = pl.multiple_of(s, _LANES)
                r = rb[pl.ds(s, _LANES)]
                g = gb[pl.ds(s, _LANES)]
                b = bb[pl.ds(s, _LANES)]
                rid = (r * _INV_BIN).astype(jnp.int32)
                gid = (g * _INV_BIN).astype(jnp.int32)
                bid = (b * _INV_BIN).astype(jnp.int32)
                rd = (r - rid.astype(jnp.float32) * _BINSIZE) * _INV_BIN
                gd = (g - gid.astype(jnp.float32) * _BINSIZE) * _INV_BIN
                bd = (b - bid.astype(jnp.float32) * _BINSIZE) * _INV_BIN
                i000 = rid + gid * _DIM + bid * (_DIM * _DIM)
                i010 = i000 + _DIM
                i001 = i000 + _DIM * _DIM
                i011 = i001 + _DIM

                def interp(coff):
                    j000 = i000 + coff if coff else i000
                    j010 = i010 + coff if coff else i010
                    j001 = i001 + coff if coff else i001
                    j011 = i011 + coff if coff else i011
                    v000 = plsc.load_gather(lut, [j000])
                    v100 = plsc.load_gather(lut, [j000 + 1])
                    v010 = plsc.load_gather(lut, [j010])
                    v110 = plsc.load_gather(lut, [j010 + 1])
                    v001 = plsc.load_gather(lut, [j001])
                    v101 = plsc.load_gather(lut, [j001 + 1])
                    v011 = plsc.load_gather(lut, [j011])
                    v111 = plsc.load_gather(lut, [j011 + 1])
                    a = v000 + rd * (v100 - v000)
                    b_ = v010 + rd * (v110 - v010)
                    c_ = v001 + rd * (v101 - v001)
                    d_ = v011 + rd * (v111 - v011)
                    e = a + gd * (b_ - a)
                    f = c_ + gd * (d_ - c_)
                    return e + bd * (f - e)

                o0[pl.ds(s, _LANES)] = interp(0)
                o1[pl.ds(s, _LANES)] = interp(_N3)
                o2[pl.ds(s, _LANES)] = interp(2 * _N3)

        start_in(0, 0)
        start_in(1, 1)

        @pl.loop(0, nchunks, step=2)
        def _pair(ci):
            for slot in range(2):
                chunk = ci + slot
                wait_in(slot)

                @pl.when(chunk >= 2)
                def _():
                    wait_out(slot)

                compute(slot)
                start_out(chunk, slot)

                @pl.when(chunk + 2 < nchunks)
                def _():
                    start_in(chunk + 2, slot)

        wait_out(0)
        wait_out(1)

    return sc_fn


def kernel(LUT, x):
    B, C, H, W = x.shape
    plane = H * W
    npix = B * plane
    lut_flat = LUT.reshape(3 * _N3)
    x_flat = x.reshape(npix * 3)
    out_flat = _build_sc_call(npix, plane)(lut_flat, x_flat)
    return out_flat.reshape(B, C, H, W)
